# Initial kernel scaffold; baseline (speedup 1.0000x reference)
import jax
import jax.numpy as jnp
from jax.experimental import pallas as pl


def kernel(input_nodes, src0, dst0, src1, dst1, table, W_neigh0, W_self0, b0, W_neigh1, W_self1, b1):
    raise NotImplementedError("write your pallas kernel here")


# speedup vs baseline: 5.1682x; 5.1682x over previous
import jax
import jax.numpy as jnp
from jax import lax
from jax.experimental import pallas as pl
from jax.experimental.pallas import tpu as pltpu
from jax.experimental.pallas import tpu_sc as plsc

_NUM_NODES = 500000
_F = 128
_N0 = 100000
_N1 = 20000
_N2 = 4096
_E0 = 320000
_E1 = 65536

_NS = 16
_C0 = 128
_NCHT = _E0 // _C0
_NCH0 = -(-_NCHT // _NS)
_HROWS = 10240
_AROWS = _HROWS + 8
_RPS = _HROWS // _NS
_N1P = 2 * _HROWS
_CS = 80
_NSCH = _N1 // _CS

_C1 = 128
_NCHT1 = _E1 // _C1
_NCH1 = _NCHT1 // _NS
_HROWS1 = _N2 // 2
_AROWS1 = _HROWS1 + 8
_RPS1 = _HROWS1 // _NS
_DR0 = _HROWS // _F
_DR1 = _HROWS1 // _F


def _deg_update(degloc, dvl16, ok, iota16):
    prev = dvl16.at[jnp.maximum(iota16 - 1, 0)].get(
        mode="promise_in_bounds")
    first = (dvl16 != prev) | (iota16 == 0)
    sm = jnp.where(first, iota16, 16)
    for k in (1, 2, 4, 8):
        sh = sm.at[jnp.minimum(iota16 + k, 15)].get(mode="promise_in_bounds")
        sm = jnp.minimum(sm, jnp.where(iota16 + k < 16, sh, 16))
    nf = sm.at[jnp.minimum(iota16 + 1, 15)].get(mode="promise_in_bounds")
    nxt = jnp.where(iota16 == 15, 16, nf)
    rl = (nxt - iota16).astype(jnp.float32)
    plsc.addupdate_scatter(degloc, [dvl16 >> 7, dvl16 & 127], rl,
                           mask=first & ok)


def _zero_vmem2d(ref, nrows):
    def zz(i, cc):
        ref[i >> 3, pl.ds((i & 7) * 16, 16)] = jnp.zeros((16,), jnp.float32)
        return cc
    lax.fori_loop(0, nrows * 8, zz, 0)


def _fill_iota(ref, n, iota16):
    def ff(r, cc):
        ref[pl.ds(r * 16, 16)] = iota16 + r * 16
        return cc
    lax.fori_loop(0, n // 16, ff, 0)


def _sc_layer0_body(inodes, src0, dst0, table, zf, zd,
                    agg, deg, hself,
                    sv, ix, dv, dvl, rows, degloc, didx, agg_sh, deg_sh, sem):
    c = lax.axis_index("c")
    s = lax.axis_index("s")
    lo_c = c * _HROWS
    garbage = _HROWS
    iota16 = lax.iota(jnp.int32, 16)

    r0 = s * _RPS
    pltpu.sync_copy(zf.at[pl.ds(r0, _RPS)], agg_sh.at[pl.ds(r0, _RPS)])

    @pl.when(s == 0)
    def _():
        pltpu.sync_copy(zd, deg_sh)

    _zero_vmem2d(degloc, _DR0)
    _fill_iota(didx, _DR0, iota16)
    plsc.subcore_barrier()

    def chunk(i, carry):
        cid = s + _NS * i

        @pl.when(cid < _NCHT)
        def _():
            base = cid * _C0
            pltpu.sync_copy(dst0.at[pl.ds(base, _C0)], dv)
            clo = dv[pl.ds(0, 16)][0]
            chi = dv[pl.ds(_C0 - 16, 16)][15]

            @pl.when((chi >= lo_c) & (clo < lo_c + _HROWS))
            def _():
                pltpu.sync_copy(src0.at[pl.ds(base, _C0)], sv)
                pltpu.async_copy(inodes.at[sv], ix, sem).wait()
                pltpu.async_copy(table.at[ix], rows, sem).wait()

                def lanes(j, cc):
                    d16 = dv[pl.ds(j * 16, 16)] - lo_c
                    ok = (d16 >= 0) & (d16 < _HROWS)
                    dvl16 = jnp.where(ok, d16, garbage)
                    dvl[pl.ds(j * 16, 16)] = dvl16
                    _deg_update(degloc, dvl16, ok, iota16)
                    return cc
                lax.fori_loop(0, _C0 // 16, lanes, 0)

                pltpu.sync_copy(rows, agg_sh.at[dvl], add=True)
        return carry
    lax.fori_loop(0, _NCH0, chunk, 0)

    w = c * _NS + s
    for k in range(8):
        cid = w + 32 * k

        @pl.when(cid < _NSCH)
        def _(cid=cid):
            base = cid * _CS
            pltpu.sync_copy(inodes.at[pl.ds(base, _CS)], ix.at[pl.ds(0, _CS)])
            pltpu.async_copy(table.at[ix.at[pl.ds(0, _CS)]],
                             rows.at[pl.ds(0, _CS)], sem).wait()
            pltpu.sync_copy(rows.at[pl.ds(0, _CS)], hself.at[pl.ds(base, _CS)])

    pltpu.sync_copy(degloc, deg_sh.at[didx], add=True)
    plsc.subcore_barrier()

    o0 = c * _HROWS + r0
    pltpu.sync_copy(agg_sh.at[pl.ds(r0, _RPS)], agg.at[pl.ds(o0, _RPS)])

    @pl.when(s < _DR0 // 8)
    def _():
        pltpu.sync_copy(deg_sh.at[pl.ds(s * 8, 8)],
                        deg.at[pl.ds(c * _DR0 + s * 8, 8)])


def _sc_layer1_body(src1, dst1, h1, zf, zd,
                    agg, deg,
                    sv, dv, dvl, rows, degloc, didx, agg_sh, deg_sh, sem):
    c = lax.axis_index("c")
    s = lax.axis_index("s")
    lo_c = c * _HROWS1
    r0 = s * _RPS1
    iota16 = lax.iota(jnp.int32, 16)

    pltpu.sync_copy(zf.at[pl.ds(r0, _RPS1)], agg_sh.at[pl.ds(r0, _RPS1)])

    @pl.when(s == 0)
    def _():
        pltpu.sync_copy(zd, deg_sh)

    _zero_vmem2d(degloc, _DR1)
    _fill_iota(didx, _DR1, iota16)
    plsc.subcore_barrier()

    def chunk(i, carry):
        cid = s + _NS * i
        base = cid * _C1
        pltpu.sync_copy(dst1.at[pl.ds(base, _C1)], dv)
        clo = dv[pl.ds(0, 16)][0]
        chi = dv[pl.ds(_C1 - 16, 16)][15]

        @pl.when((chi >= lo_c) & (clo < lo_c + _HROWS1))
        def _():
            pltpu.sync_copy(src1.at[pl.ds(base, _C1)], sv)
            pltpu.async_copy(h1.at[sv], rows, sem).wait()

            def lanes(j, cc):
                d16 = dv[pl.ds(j * 16, 16)] - lo_c
                ok = (d16 >= 0) & (d16 < _HROWS1)
                dvl16 = jnp.where(ok, d16, _HROWS1)
                dvl[pl.ds(j * 16, 16)] = dvl16
                _deg_update(degloc, dvl16, ok, iota16)
                return cc
            lax.fori_loop(0, _C1 // 16, lanes, 0)

            pltpu.sync_copy(rows, agg_sh.at[dvl], add=True)
        return carry
    lax.fori_loop(0, _NCH1, chunk, 0)

    pltpu.sync_copy(degloc, deg_sh.at[didx], add=True)
    plsc.subcore_barrier()

    o0 = c * _HROWS1 + r0
    pltpu.sync_copy(agg_sh.at[pl.ds(r0, _RPS1)], agg.at[pl.ds(o0, _RPS1)])

    @pl.when(s < _DR1 // 8)
    def _():
        pltpu.sync_copy(deg_sh.at[pl.ds(s * 8, 8)],
                        deg.at[pl.ds(c * _DR1 + s * 8, 8)])


def _tc_layer0(agg, deg, hs, wn, ws, bb, out):
    rec = 1.0 / jnp.maximum(deg[...], 1.0)
    acc = jnp.dot(agg[...] * rec, wn[...], preferred_element_type=jnp.float32)
    acc += jnp.dot(hs[...], ws[...], preferred_element_type=jnp.float32)
    out[...] = jnp.maximum(acc + bb[...], 0.0)


def _tc_layer1(h1s, agg, deg, wn, ws, bb, out):
    rec = 1.0 / jnp.maximum(deg[...], 1.0)
    neigh = agg[...] * rec
    acc = jnp.dot(h1s[...], ws[...], preferred_element_type=jnp.float32)
    acc += jnp.dot(neigh, wn[...], preferred_element_type=jnp.float32)
    out[...] = acc + bb[...]


def kernel(input_nodes, src0, dst0, src1, dst1, table,
           W_neigh0, W_self0, b0, W_neigh1, W_self1, b1):
    f32 = jnp.float32
    mesh = plsc.VectorSubcoreMesh(core_axis_name="c", subcore_axis_name="s",
                                  num_cores=2, num_subcores=_NS)

    zf0 = jnp.zeros((_HROWS, _F), f32)
    zd0 = jnp.zeros((_DR0, _F), f32)

    sc0 = pl.kernel(
        _sc_layer0_body,
        out_type=[
            jax.ShapeDtypeStruct((_N1P, _F), f32),
            jax.ShapeDtypeStruct((2 * _DR0, _F), f32),
            jax.ShapeDtypeStruct((_N1, _F), f32),
        ],
        mesh=mesh,
        compiler_params=pltpu.CompilerParams(needs_layout_passes=False),
        scratch_types=[
            pltpu.VMEM((_C0,), jnp.int32),
            pltpu.VMEM((_C0,), jnp.int32),
            pltpu.VMEM((_C0,), jnp.int32),
            pltpu.VMEM((_C0,), jnp.int32),
            pltpu.VMEM((_C0, _F), f32),
            pltpu.VMEM((_DR0, _F), f32),
            pltpu.VMEM((_DR0,), jnp.int32),
            pltpu.VMEM_SHARED((_AROWS, _F), f32),
            pltpu.VMEM_SHARED((_DR0, _F), f32),
            pltpu.SemaphoreType.DMA,
        ],
    )
    agg0, deg0, hself = sc0(input_nodes, src0, dst0, table, zf0, zd0)

    bn = _N1 // 10
    h1 = pl.pallas_call(
        _tc_layer0,
        grid=(10,),
        in_specs=[
            pl.BlockSpec((bn, _F), lambda i: (i, 0)),
            pl.BlockSpec((bn, 1), lambda i: (i, 0)),
            pl.BlockSpec((bn, _F), lambda i: (i, 0)),
            pl.BlockSpec((_F, _F), lambda i: (0, 0)),
            pl.BlockSpec((_F, _F), lambda i: (0, 0)),
            pl.BlockSpec((1, _F), lambda i: (0, 0)),
        ],
        out_specs=pl.BlockSpec((bn, _F), lambda i: (i, 0)),
        out_shape=jax.ShapeDtypeStruct((_N1, _F), f32),
    )(agg0, deg0.reshape(_N1P, 1), hself, W_neigh0, W_self0,
      b0.reshape(1, _F))

    zf1 = jnp.zeros((_HROWS1, _F), f32)
    zd1 = jnp.zeros((_DR1, _F), f32)

    sc1 = pl.kernel(
        _sc_layer1_body,
        out_type=[
            jax.ShapeDtypeStruct((_N2, _F), f32),
            jax.ShapeDtypeStruct((2 * _DR1, _F), f32),
        ],
        mesh=mesh,
        compiler_params=pltpu.CompilerParams(needs_layout_passes=False),
        scratch_types=[
            pltpu.VMEM((_C1,), jnp.int32),
            pltpu.VMEM((_C1,), jnp.int32),
            pltpu.VMEM((_C1,), jnp.int32),
            pltpu.VMEM((_C1, _F), f32),
            pltpu.VMEM((_DR1, _F), f32),
            pltpu.VMEM((_DR1,), jnp.int32),
            pltpu.VMEM_SHARED((_AROWS1, _F), f32),
            pltpu.VMEM_SHARED((_DR1, _F), f32),
            pltpu.SemaphoreType.DMA,
        ],
    )
    agg1, deg1 = sc1(src1, dst1, h1, zf1, zd1)

    wn1 = jnp.pad(W_neigh1, ((0, 0), (0, _F - 47)))
    ws1 = jnp.pad(W_self1, ((0, 0), (0, _F - 47)))
    b1p = jnp.pad(b1, (0, _F - 47)).reshape(1, _F)

    out_p = pl.pallas_call(
        _tc_layer1,
        out_shape=jax.ShapeDtypeStruct((_N2, _F), f32),
    )(h1[:_N2], agg1, deg1.reshape(_N2, 1), wn1, ws1, b1p)
    return out_p[:, :47]
